# Initial kernel scaffold; baseline (speedup 1.0000x reference)
import jax
import jax.numpy as jnp
from jax.experimental import pallas as pl


def kernel(x, edge_attr, EigVals, EigVecs, atom_tables, bond_tables, A_W, A_b, B_W, B_b):
    raise NotImplementedError("write your pallas kernel here")


# speedup vs baseline: 6.6462x; 6.6462x over previous
import functools

import jax
import jax.numpy as jnp
from jax import lax
from jax.experimental import pallas as pl

ATOM_DIMS = (119, 4, 12, 12, 10, 6, 6, 2, 2)
BOND_DIMS = (5, 6, 2)
ATOM_K = sum(ATOM_DIMS)
BOND_K = sum(BOND_DIMS)
HIDDEN = 256
LAP = 16
NFREQ = 16

_ATOM_OFFS = tuple(sum(ATOM_DIMS[:i]) for i in range(len(ATOM_DIMS)))
_BOND_OFFS = tuple(sum(BOND_DIMS[:i]) for i in range(len(BOND_DIMS)))


def _node_body(x_ref, ev_ref, el_ref, atab_ref, aw_ref, ab_ref, bw_ref,
               bb_ref, out_ref):
    B = x_ref.shape[0]
    xs = x_ref[...]
    kio = lax.broadcasted_iota(jnp.int32, (B, ATOM_K), 1)
    oh = jnp.zeros((B, ATOM_K), jnp.float32)
    for j, off in enumerate(_ATOM_OFFS):
        oh = oh + (kio == (xs[:, j:j + 1] + off)).astype(jnp.float32)
    h = lax.dot_general(oh, atab_ref[...], (((1,), (0,)), ((), ())),
                        preferred_element_type=jnp.float32)

    ev = ev_ref[...]
    el = el_ref[...]
    mv = jnp.isnan(ev)
    evc = jnp.where(mv, 0.0, ev)
    elc = jnp.where(jnp.isnan(el), 0.0, el)
    aw = aw_ref[...]
    ab = ab_ref[...]
    bw = bw_ref[...]
    bb = bb_ref[...]
    acc = jnp.zeros((B, LAP), jnp.float32)
    for f in range(NFREQ):
        pe1 = jnp.maximum(
            evc[:, f:f + 1] * aw[0:1, :] + elc[:, f:f + 1] * aw[1:2, :] + ab,
            0.0)
        pe2 = jnp.maximum(
            lax.dot_general(pe1, bw, (((1,), (0,)), ((), ())),
                            preferred_element_type=jnp.float32) + bb,
            0.0)
        acc = acc + jnp.where(mv[:, f:f + 1], 0.0, pe2)
    out_ref[...] = jnp.concatenate([h, acc], axis=1)


def _edge_body(ea_ref, btab_ref, out_ref):
    B = ea_ref.shape[0]
    ea = ea_ref[...]
    kio = lax.broadcasted_iota(jnp.int32, (B, BOND_K), 1)
    oh = jnp.zeros((B, BOND_K), jnp.float32)
    for j, off in enumerate(_BOND_OFFS):
        oh = oh + (kio == (ea[:, j:j + 1] + off)).astype(jnp.float32)
    out_ref[...] = lax.dot_general(oh, btab_ref[...], (((1,), (0,)), ((), ())),
                                   preferred_element_type=jnp.float32)


def kernel(x, edge_attr, EigVals, EigVecs, atom_tables, bond_tables,
           A_W, A_b, B_W, B_b):
    N = x.shape[0]
    E = edge_attr.shape[0]
    atab = jnp.concatenate(atom_tables, axis=0)
    btab = jnp.concatenate(bond_tables, axis=0)
    el2 = EigVals[:, :, 0]
    ab2 = A_b.reshape(1, -1)
    bb2 = B_b.reshape(1, -1)

    BN = 1000
    rep = lambda shape: pl.BlockSpec(shape, lambda i: (0,) * len(shape))
    node_out = pl.pallas_call(
        _node_body,
        grid=(N // BN,),
        in_specs=[
            pl.BlockSpec((BN, x.shape[1]), lambda i: (i, 0)),
            pl.BlockSpec((BN, NFREQ), lambda i: (i, 0)),
            pl.BlockSpec((BN, NFREQ), lambda i: (i, 0)),
            rep(atab.shape),
            rep(A_W.shape),
            rep(ab2.shape),
            rep(B_W.shape),
            rep(bb2.shape),
        ],
        out_specs=pl.BlockSpec((BN, HIDDEN), lambda i: (i, 0)),
        out_shape=jax.ShapeDtypeStruct((N, HIDDEN), jnp.float32),
    )(x, EigVecs, el2, atab, A_W, ab2, B_W, bb2)

    BE = 4000
    e_out = pl.pallas_call(
        _edge_body,
        grid=(E // BE,),
        in_specs=[
            pl.BlockSpec((BE, edge_attr.shape[1]), lambda i: (i, 0)),
            rep(btab.shape),
        ],
        out_specs=pl.BlockSpec((BE, HIDDEN), lambda i: (i, 0)),
        out_shape=jax.ShapeDtypeStruct((E, HIDDEN), jnp.float32),
    )(edge_attr, btab)
    return node_out, e_out
